# baseline (device time: 254499 ns/iter reference)
import jax
import jax.numpy as jnp
from jax import lax
from jax.experimental import pallas as pl
from jax.experimental.pallas import tpu as pltpu

M, N = 4096, 2048
CHUNKS = 8
CC = N // CHUNKS

BANDS = (
    (0, 1408, ("x", "y", "z")),
    (1408, 1344, ("y", "z", "x")),
    (2752, 1344, ("z", "x", "y")),
)


def kernel(x):
    x = x.reshape(M, N)

    def body(x_ref, out_ref, comm0, comm1, comm2,
             rs_send, rs_recv, ag_send, ag_recv, load_sems):
        comms = (comm0, comm1, comm2)
        coord = {
            "x": lax.axis_index("x"),
            "y": lax.axis_index("y"),
            "z": lax.axis_index("z"),
        }

        def partner(d):
            return tuple(
                1 - coord[a] if a == d else coord[a] for a in ("x", "y", "z")
            )

        offs, send_offs, parts = [], [], []
        for base, R, dims in BANDS:
            o = [base]
            so = []
            pt = []
            for s in range(3):
                h = R >> (s + 1)
                bit = coord[dims[s]]
                so.append(o[s] + (1 - bit) * h)
                o.append(o[s] + bit * h)
                pt.append(partner(dims[s]))
            offs.append(o)
            send_offs.append(so)
            parts.append(pt)
        cums = [(0, R >> 1, (R >> 1) + (R >> 2)) for (_, R, _) in BANDS]

        loads = []
        for b, (base, R, dims) in enumerate(BANDS):
            rows = pl.ds(offs[b][1], R >> 1)
            ld = pltpu.make_async_copy(
                x_ref.at[rows, :], out_ref.at[rows, :], load_sems.at[b],
            )
            ld.start()
            loads.append(ld)

        barrier_sem = pltpu.get_barrier_semaphore()
        for d in ("x", "y", "z"):
            pl.semaphore_signal(
                barrier_sem, inc=1,
                device_id=partner(d), device_id_type=pl.DeviceIdType.MESH,
            )
        pl.semaphore_wait(barrier_sem, 3)

        def rs_rdma(b, s, c):
            h = BANDS[b][1] >> (s + 1)
            cols = pl.ds(c * CC, CC)
            src = x_ref if s == 0 else out_ref
            return pltpu.make_async_remote_copy(
                src_ref=src.at[pl.ds(send_offs[b][s], h), cols],
                dst_ref=comms[b].at[pl.ds(cums[b][s], h), cols],
                send_sem=rs_send.at[b, s, c],
                recv_sem=rs_recv.at[b, s, c],
                device_id=parts[b][s],
                device_id_type=pl.DeviceIdType.MESH,
            )

        def ag_rdma(b, s, c):
            h = BANDS[b][1] >> (s + 1)
            rows = pl.ds(offs[b][s + 1], h)
            cols = pl.ds(c * CC, CC)
            return pltpu.make_async_remote_copy(
                src_ref=out_ref.at[rows, cols],
                dst_ref=out_ref.at[rows, cols],
                send_sem=ag_send.at[b, s, c],
                recv_sem=ag_recv.at[b, s, c],
                device_id=parts[b][s],
                device_id_type=pl.DeviceIdType.MESH,
            )

        rdmas = [[None] * CHUNKS for _ in range(3)]
        for b in range(3):
            for c in range(CHUNKS):
                r = rs_rdma(b, 0, c)
                r.start()
                rdmas[b][c] = r
        ag_rdmas = [[None] * CHUNKS for _ in range(3)]
        for s in range(3):
            for c in range(CHUNKS):
                for b in range(3):
                    rdmas[b][c].wait()
                    if s == 0 and c == 0:
                        loads[b].wait()
                    h = BANDS[b][1] >> (s + 1)
                    hn = h >> 1
                    kept = offs[b][s + 1]
                    cbase = cums[b][s]
                    cols = pl.ds(c * CC, CC)
                    if s < 2:
                        j = send_offs[b][s + 1] - kept
                        rows = pl.ds(send_offs[b][s + 1], hn)
                        out_ref[rows, cols] = (
                            out_ref[rows, cols]
                            + comms[b][pl.ds(cbase + j, hn), cols]
                        )
                        nxt = rs_rdma(b, s + 1, c)
                        nxt.start()
                        rdmas[b][c] = nxt
                        jk = offs[b][s + 2] - kept
                        rows = pl.ds(offs[b][s + 2], hn)
                        out_ref[rows, cols] = (
                            out_ref[rows, cols]
                            + comms[b][pl.ds(cbase + jk, hn), cols]
                        )
                    else:
                        rows = pl.ds(offs[b][3], h)
                        out_ref[rows, cols] = (
                            out_ref[rows, cols]
                            + comms[b][pl.ds(cbase, h), cols]
                        )
                        r = ag_rdma(b, 2, c)
                        r.start()
                        ag_rdmas[b][c] = r

        for s in (2, 1, 0):
            for c in range(CHUNKS):
                for b in range(3):
                    ag_rdmas[b][c].wait()
                    if s > 0:
                        nxt = ag_rdma(b, s - 1, c)
                        nxt.start()
                        ag_rdmas[b][c] = nxt

    return pl.pallas_call(
        body,
        out_shape=jax.ShapeDtypeStruct((M, N), jnp.float32),
        in_specs=[pl.BlockSpec(memory_space=pl.ANY)],
        out_specs=pl.BlockSpec(memory_space=pltpu.VMEM),
        scratch_shapes=[
            pltpu.VMEM((BANDS[0][1] * 7 // 8, N), jnp.float32),
            pltpu.VMEM((BANDS[1][1] * 7 // 8, N), jnp.float32),
            pltpu.VMEM((BANDS[2][1] * 7 // 8, N), jnp.float32),
            pltpu.SemaphoreType.DMA((3, 3, CHUNKS)),
            pltpu.SemaphoreType.DMA((3, 3, CHUNKS)),
            pltpu.SemaphoreType.DMA((3, 3, CHUNKS)),
            pltpu.SemaphoreType.DMA((3, 3, CHUNKS)),
            pltpu.SemaphoreType.DMA((3,)),
        ],
        compiler_params=pltpu.CompilerParams(
            collective_id=0,
            vmem_limit_bytes=63 * 1024 * 1024,
        ),
    )(x)


# device time: 245031 ns/iter; 1.0386x vs baseline; 1.0386x over previous
import jax
import jax.numpy as jnp
from jax import lax
from jax.experimental import pallas as pl
from jax.experimental.pallas import tpu as pltpu

M, N = 4096, 2048
CHUNKS = 4
CC = N // CHUNKS

BANDS = (
    (0, 1408, ("x", "y", "z")),
    (1408, 1344, ("y", "z", "x")),
    (2752, 1344, ("z", "x", "y")),
)


def kernel(x):
    x = x.reshape(M, N)

    def body(x_ref, out_ref, acc, comm0, comm1, comm2,
             rs_send, rs_recv, ag_send, ag_recv, load_sems, store_sems):
        comms = (comm0, comm1, comm2)
        coord = {
            "x": lax.axis_index("x"),
            "y": lax.axis_index("y"),
            "z": lax.axis_index("z"),
        }

        def partner(d):
            return tuple(
                1 - coord[a] if a == d else coord[a] for a in ("x", "y", "z")
            )

        offs, send_offs, parts = [], [], []
        for base, R, dims in BANDS:
            o = [base]
            so = []
            pt = []
            for s in range(3):
                h = R >> (s + 1)
                bit = coord[dims[s]]
                so.append(o[s] + (1 - bit) * h)
                o.append(o[s] + bit * h)
                pt.append(partner(dims[s]))
            offs.append(o)
            send_offs.append(so)
            parts.append(pt)
        cums = [(0, R >> 1, (R >> 1) + (R >> 2)) for (_, R, _) in BANDS]

        loads = []
        for b, (base, R, dims) in enumerate(BANDS):
            rows = pl.ds(offs[b][1], R >> 1)
            ld = pltpu.make_async_copy(
                x_ref.at[rows, :], acc.at[rows, :], load_sems.at[b],
            )
            ld.start()
            loads.append(ld)

        barrier_sem = pltpu.get_barrier_semaphore()
        for d in ("x", "y", "z"):
            pl.semaphore_signal(
                barrier_sem, inc=1,
                device_id=partner(d), device_id_type=pl.DeviceIdType.MESH,
            )
        pl.semaphore_wait(barrier_sem, 3)

        def rs_rdma(b, s, c):
            h = BANDS[b][1] >> (s + 1)
            cols = pl.ds(c * CC, CC)
            src = x_ref if s == 0 else acc
            return pltpu.make_async_remote_copy(
                src_ref=src.at[pl.ds(send_offs[b][s], h), cols],
                dst_ref=comms[b].at[pl.ds(cums[b][s], h), cols],
                send_sem=rs_send.at[b, s, c],
                recv_sem=rs_recv.at[b, s, c],
                device_id=parts[b][s],
                device_id_type=pl.DeviceIdType.MESH,
            )

        def ag_rdma(b, s, c):
            h = BANDS[b][1] >> (s + 1)
            rows = pl.ds(offs[b][s + 1], h)
            cols = pl.ds(c * CC, CC)
            return pltpu.make_async_remote_copy(
                src_ref=acc.at[rows, cols],
                dst_ref=acc.at[rows, cols],
                send_sem=ag_send.at[b, s, c],
                recv_sem=ag_recv.at[b, s, c],
                device_id=parts[b][s],
                device_id_type=pl.DeviceIdType.MESH,
            )

        stores = []

        def store(b, t, c, roff, h):
            rows = pl.ds(roff, h)
            cols = pl.ds(c * CC, CC)
            st = pltpu.make_async_copy(
                acc.at[rows, cols], out_ref.at[rows, cols],
                store_sems.at[b, t, c],
            )
            st.start()
            stores.append(st)

        rdmas = [[None] * CHUNKS for _ in range(3)]
        for b in range(3):
            for c in range(CHUNKS):
                r = rs_rdma(b, 0, c)
                r.start()
                rdmas[b][c] = r
        ag_rdmas = [[None] * CHUNKS for _ in range(3)]
        for s in range(3):
            for c in range(CHUNKS):
                for b in range(3):
                    rdmas[b][c].wait()
                    if s == 0 and c == 0:
                        loads[b].wait()
                    h = BANDS[b][1] >> (s + 1)
                    hn = h >> 1
                    kept = offs[b][s + 1]
                    cbase = cums[b][s]
                    cols = pl.ds(c * CC, CC)
                    if s < 2:
                        j = send_offs[b][s + 1] - kept
                        rows = pl.ds(send_offs[b][s + 1], hn)
                        acc[rows, cols] = (
                            acc[rows, cols]
                            + comms[b][pl.ds(cbase + j, hn), cols]
                        )
                        nxt = rs_rdma(b, s + 1, c)
                        nxt.start()
                        rdmas[b][c] = nxt
                        jk = offs[b][s + 2] - kept
                        rows = pl.ds(offs[b][s + 2], hn)
                        acc[rows, cols] = (
                            acc[rows, cols]
                            + comms[b][pl.ds(cbase + jk, hn), cols]
                        )
                    else:
                        rows = pl.ds(offs[b][3], h)
                        acc[rows, cols] = (
                            acc[rows, cols]
                            + comms[b][pl.ds(cbase, h), cols]
                        )
                        r = ag_rdma(b, 2, c)
                        r.start()
                        ag_rdmas[b][c] = r
                        store(b, 3, c, offs[b][3], h)

        for s in (2, 1, 0):
            for c in range(CHUNKS):
                for b in range(3):
                    ag_rdmas[b][c].wait()
                    if s > 0:
                        nxt = ag_rdma(b, s - 1, c)
                        nxt.start()
                        ag_rdmas[b][c] = nxt
                    h = BANDS[b][1] >> (s + 1)
                    recv_off = 2 * offs[b][s] + h - offs[b][s + 1]
                    store(b, s, c, recv_off, h)

        for st in stores:
            st.wait()

    return pl.pallas_call(
        body,
        out_shape=jax.ShapeDtypeStruct((M, N), jnp.float32),
        in_specs=[pl.BlockSpec(memory_space=pl.ANY)],
        out_specs=pl.BlockSpec(memory_space=pl.ANY),
        scratch_shapes=[
            pltpu.VMEM((M, N), jnp.float32),
            pltpu.VMEM((BANDS[0][1] * 7 // 8, N), jnp.float32),
            pltpu.VMEM((BANDS[1][1] * 7 // 8, N), jnp.float32),
            pltpu.VMEM((BANDS[2][1] * 7 // 8, N), jnp.float32),
            pltpu.SemaphoreType.DMA((3, 3, CHUNKS)),
            pltpu.SemaphoreType.DMA((3, 3, CHUNKS)),
            pltpu.SemaphoreType.DMA((3, 3, CHUNKS)),
            pltpu.SemaphoreType.DMA((3, 3, CHUNKS)),
            pltpu.SemaphoreType.DMA((3,)),
            pltpu.SemaphoreType.DMA((3, 4, CHUNKS)),
        ],
        compiler_params=pltpu.CompilerParams(
            collective_id=0,
            vmem_limit_bytes=63 * 1024 * 1024,
        ),
    )(x)
